# Initial kernel scaffold; baseline (speedup 1.0000x reference)
import jax
import jax.numpy as jnp
from jax.experimental import pallas as pl


def kernel(seed_maps, emb_maps, labels, instances):
    raise NotImplementedError("write your pallas kernel here")


# speedup vs baseline: 53.3689x; 53.3689x over previous
import jax
import jax.numpy as jnp
from jax import lax
from jax.experimental import pallas as pl
from jax.experimental.pallas import tpu as pltpu

_B, _NC, _C, _H, _W = 4, 2, 8, 512, 512
_NI = 16
_RAD = 10.0
_W_INST, _W_VAR, _W_SEED = 1.0, 10.0, 1.0
_BIG = 1e9

_HB = 64
_NB = _H // _HB
_P = _HB * _W


def _stats_body(emb_ref, inst_ref, kf_ref, rect_ref, sums_s, cnt_s, bbox_s):
    i = pl.program_id(1)

    @pl.when(i == 0)
    def _init():
        sums_s[...] = jnp.zeros_like(sums_s)
        cnt_s[...] = jnp.zeros_like(cnt_s)
        col = lax.broadcasted_iota(jnp.int32, (_NI, 4), 1)
        bbox_s[...] = jnp.where((col == 0) | (col == 2), _BIG, -_BIG)

    emb = emb_ref[0].reshape(_C, _P)
    inst = inst_ref[0, 0].reshape(1, _P)
    ids = lax.broadcasted_iota(jnp.int32, (_NI, 1), 0)
    mask = inst == ids
    maskf = mask.astype(jnp.float32)

    sums_s[...] += lax.dot_general(
        maskf, emb, (((1,), (1,)), ((), ())),
        preferred_element_type=jnp.float32)
    cnt_s[...] += jnp.sum(maskf, axis=1, keepdims=True)

    flat = lax.broadcasted_iota(jnp.int32, (1, _P), 1)
    x = (flat % _W).astype(jnp.float32)
    y = (flat // _W + i * _HB).astype(jnp.float32)
    minx = jnp.min(jnp.where(mask, x, _BIG), axis=1, keepdims=True)
    maxx = jnp.max(jnp.where(mask, x, -_BIG), axis=1, keepdims=True)
    miny = jnp.min(jnp.where(mask, y, _BIG), axis=1, keepdims=True)
    maxy = jnp.max(jnp.where(mask, y, -_BIG), axis=1, keepdims=True)
    bbox_s[:, 0:1] = jnp.minimum(bbox_s[:, 0:1], minx)
    bbox_s[:, 1:2] = jnp.maximum(bbox_s[:, 1:2], maxx)
    bbox_s[:, 2:3] = jnp.minimum(bbox_s[:, 2:3], miny)
    bbox_s[:, 3:4] = jnp.maximum(bbox_s[:, 3:4], maxy)

    @pl.when(i == _NB - 1)
    def _fin():
        cnt = cnt_s[...]
        keyf = sums_s[...] / cnt
        nrm = jnp.sqrt(jnp.sum(keyf * keyf, axis=1, keepdims=True))
        kf_ref[0] = keyf / jnp.maximum(nrm, 1e-12)
        minx_ = bbox_s[:, 0:1]
        maxx_ = bbox_s[:, 1:2]
        miny_ = bbox_s[:, 2:3]
        maxy_ = bbox_s[:, 3:4]
        x1 = jnp.where(minx_ > _RAD, minx_ - _RAD, 0.0)
        x2 = jnp.where(maxx_ < _W - _RAD, maxx_ + _RAD, float(_W))
        y1 = jnp.where(miny_ > _RAD, miny_ - _RAD, 0.0)
        y2 = jnp.where(maxy_ < _H - _RAD, maxy_ + _RAD, float(_H))
        rect_ref[0] = jnp.concatenate([x1, x2, y1, y2], axis=1)


def _loss_body(emb_ref, inst_ref, seed_ref, lbl_ref, kf_ref, rect_ref,
               out_ref, cmat_s, intra_s, focal_s, acc_s):
    b = pl.program_id(0)
    i = pl.program_id(1)

    @pl.when((b == 0) & (i == 0))
    def _init_all():
        focal_s[0, 0] = 0.0
        acc_s[0, 0] = 0.0

    @pl.when(i == 0)
    def _init_b():
        cmat_s[...] = jnp.zeros_like(cmat_s)
        intra_s[0, 0] = 0.0

    emb = emb_ref[0].reshape(_C, _P)
    inst = inst_ref[0, 0].reshape(1, _P)
    ids = lax.broadcasted_iota(jnp.int32, (_NI, 1), 0)
    maskf = (inst == ids).astype(jnp.float32)

    kf = kf_ref[0]
    kf_pix = lax.dot_general(
        kf, maskf, (((0,), (0,)), ((), ())),
        preferred_element_type=jnp.float32)
    dotp = jnp.sum(kf_pix * emb, axis=0, keepdims=True)
    na = jnp.sqrt(jnp.sum(kf_pix * kf_pix, axis=0, keepdims=True))
    nb = jnp.sqrt(jnp.sum(emb * emb, axis=0, keepdims=True))
    cos = dotp / (jnp.maximum(na, 1e-8) * jnp.maximum(nb, 1e-8))
    intra_s[0, 0] += jnp.sum(jnp.exp(1.0 - cos) - 1.0)

    rect = rect_ref[0]
    x1 = rect[:, 0:1]
    x2 = rect[:, 1:2]
    y1 = rect[:, 2:3]
    y2 = rect[:, 3:4]
    flat = lax.broadcasted_iota(jnp.int32, (1, _P), 1)
    x = (flat % _W).astype(jnp.float32)
    y = (flat // _W + i * _HB).astype(jnp.float32)
    win = ((y >= y1) & (y < y2) & (x >= x1) & (x < x2)).astype(jnp.float32)
    cmat_s[...] += lax.dot_general(
        win, maskf, (((1,), (1,)), ((), ())),
        preferred_element_type=jnp.float32)

    s0 = seed_ref[0, 0]
    s1 = seed_ref[0, 1]
    lbl = lbl_ref[0]
    m = jnp.maximum(s0, s1)
    lse = m + jnp.log(jnp.exp(s0 - m) + jnp.exp(s1 - m))
    lpt = jnp.where(lbl == 0, s0, s1) - lse
    pt = jnp.exp(lpt)
    focal_s[0, 0] += jnp.sum((1.0 - pt) * (1.0 - pt) * (-lpt))

    @pl.when(i == _NB - 1)
    def _fin():
        kfv = kf_ref[0]
        norms = jnp.maximum(
            jnp.sqrt(jnp.sum(kfv * kfv, axis=1, keepdims=True)), 1e-8)
        gram = lax.dot_general(
            kfv, kfv, (((1,), (1,)), ((), ())),
            preferred_element_type=jnp.float32)
        outer = lax.dot_general(
            norms, norms, (((1,), (1,)), ((), ())),
            preferred_element_type=jnp.float32)
        s_abs = jnp.abs(gram / outer)

        present = (cmat_s[...] > 0.0).astype(jnp.float32)
        ki = lax.broadcasted_iota(jnp.int32, (_NI, _NI), 0)
        ji = lax.broadcasted_iota(jnp.int32, (_NI, _NI), 1)
        upper = (ki > ji).astype(jnp.float32)
        suf = lax.dot_general(
            present, upper, (((1,), (0,)), ((), ())),
            preferred_element_type=jnp.float32)
        rowm = jnp.where((ji >= 1) & (suf <= 9.0), present, 0.0)
        nm = jnp.where((ki == 0) | (ji == 0), 0.5, rowm)
        nm = jnp.where((ki == 0) & (ji == 0), 0.0, nm)
        inter = jnp.sum((jnp.exp(s_abs) - 1.0) * nm) / jnp.sum(nm)

        acc_s[0, 0] += inter * _W_INST + \
            (intra_s[0, 0] / float(_H * _W)) * _W_VAR
        out_ref[0, 0] = focal_s[0, 0] / float(_B * _H * _W) * _W_SEED + \
            acc_s[0, 0] / float(_B)


def kernel(seed_maps, emb_maps, labels, instances):
    kf, rect = pl.pallas_call(
        _stats_body,
        grid=(_B, _NB),
        in_specs=[
            pl.BlockSpec((1, _C, _HB, _W), lambda b, i: (b, 0, i, 0)),
            pl.BlockSpec((1, 1, _HB, _W), lambda b, i: (b, 0, i, 0)),
        ],
        out_specs=[
            pl.BlockSpec((1, _NI, _C), lambda b, i: (b, 0, 0)),
            pl.BlockSpec((1, _NI, 4), lambda b, i: (b, 0, 0)),
        ],
        out_shape=[
            jax.ShapeDtypeStruct((_B, _NI, _C), jnp.float32),
            jax.ShapeDtypeStruct((_B, _NI, 4), jnp.float32),
        ],
        scratch_shapes=[
            pltpu.VMEM((_NI, _C), jnp.float32),
            pltpu.VMEM((_NI, 1), jnp.float32),
            pltpu.VMEM((_NI, 4), jnp.float32),
        ],
    )(emb_maps, instances)

    out = pl.pallas_call(
        _loss_body,
        grid=(_B, _NB),
        in_specs=[
            pl.BlockSpec((1, _C, _HB, _W), lambda b, i: (b, 0, i, 0)),
            pl.BlockSpec((1, 1, _HB, _W), lambda b, i: (b, 0, i, 0)),
            pl.BlockSpec((1, _NC, _HB, _W), lambda b, i: (b, 0, i, 0)),
            pl.BlockSpec((1, _HB, _W), lambda b, i: (b, i, 0)),
            pl.BlockSpec((1, _NI, _C), lambda b, i: (b, 0, 0)),
            pl.BlockSpec((1, _NI, 4), lambda b, i: (b, 0, 0)),
        ],
        out_specs=pl.BlockSpec(memory_space=pltpu.SMEM),
        out_shape=jax.ShapeDtypeStruct((1, 1), jnp.float32),
        scratch_shapes=[
            pltpu.VMEM((_NI, _NI), jnp.float32),
            pltpu.SMEM((1, 1), jnp.float32),
            pltpu.SMEM((1, 1), jnp.float32),
            pltpu.SMEM((1, 1), jnp.float32),
        ],
    )(emb_maps, instances, seed_maps, labels, kf, rect)

    return out[0, 0]


# speedup vs baseline: 56.6794x; 1.0620x over previous
import jax
import jax.numpy as jnp
from jax import lax
from jax.experimental import pallas as pl
from jax.experimental.pallas import tpu as pltpu

_B, _NC, _C, _H, _W = 4, 2, 8, 512, 512
_NI = 16
_RAD = 10.0
_W_INST, _W_VAR, _W_SEED = 1.0, 10.0, 1.0
_BIG = 1e9

_HB = 128
_NB = _H // _HB
_P = _HB * _W


def _stats_body(emb_ref, inst_ref, kf_ref, rect_ref,
                sums_s, cnt_s, bbox_s):
    i = pl.program_id(1)

    @pl.when(i == 0)
    def _init():
        sums_s[...] = jnp.zeros_like(sums_s)
        cnt_s[...] = jnp.zeros_like(cnt_s)
        col = lax.broadcasted_iota(jnp.int32, (_NI, 4), 1)
        bbox_s[...] = jnp.where((col == 0) | (col == 2), _BIG, -_BIG)

    emb = emb_ref[0].reshape(_C, _P)
    inst = inst_ref[0, 0].reshape(1, _P)
    ids = lax.broadcasted_iota(jnp.int32, (_NI, 1), 0)
    maskf = (inst == ids).astype(jnp.float32)

    sums_s[...] += lax.dot_general(
        maskf, emb, (((1,), (1,)), ((), ())),
        preferred_element_type=jnp.float32)

    mask3 = maskf.reshape(_NI, _HB, _W)
    colc = jnp.sum(mask3, axis=1)
    rowc = jnp.sum(mask3, axis=2)
    cnt_s[...] += jnp.sum(colc, axis=1, keepdims=True)

    xi = lax.broadcasted_iota(jnp.int32, (_NI, _W), 1).astype(jnp.float32)
    yi = (lax.broadcasted_iota(jnp.int32, (_NI, _HB), 1)
          + i * _HB).astype(jnp.float32)
    bbox_s[:, 0:1] = jnp.minimum(
        bbox_s[:, 0:1],
        jnp.min(jnp.where(colc > 0, xi, _BIG), axis=1, keepdims=True))
    bbox_s[:, 1:2] = jnp.maximum(
        bbox_s[:, 1:2],
        jnp.max(jnp.where(colc > 0, xi, -_BIG), axis=1, keepdims=True))
    bbox_s[:, 2:3] = jnp.minimum(
        bbox_s[:, 2:3],
        jnp.min(jnp.where(rowc > 0, yi, _BIG), axis=1, keepdims=True))
    bbox_s[:, 3:4] = jnp.maximum(
        bbox_s[:, 3:4],
        jnp.max(jnp.where(rowc > 0, yi, -_BIG), axis=1, keepdims=True))

    @pl.when(i == _NB - 1)
    def _fin():
        cnt = cnt_s[...]
        keyf = sums_s[...] / cnt
        nrm = jnp.sqrt(jnp.sum(keyf * keyf, axis=1, keepdims=True))
        kf_ref[0] = keyf / jnp.maximum(nrm, 1e-12)
        minx_ = bbox_s[:, 0:1]
        maxx_ = bbox_s[:, 1:2]
        miny_ = bbox_s[:, 2:3]
        maxy_ = bbox_s[:, 3:4]
        x1 = jnp.where(minx_ > _RAD, minx_ - _RAD, 0.0)
        x2 = jnp.where(maxx_ < _W - _RAD, maxx_ + _RAD, float(_W))
        y1 = jnp.where(miny_ > _RAD, miny_ - _RAD, 0.0)
        y2 = jnp.where(maxy_ < _H - _RAD, maxy_ + _RAD, float(_H))
        rect_ref[0] = jnp.concatenate([x1, x2, y1, y2], axis=1)


def _loss_body(emb_ref, inst_ref, seed_ref, lbl_ref, kf_ref, rect_ref,
               out_ref, cmat_s, intra_s, focal_s, acc_s):
    b = pl.program_id(0)
    i = pl.program_id(1)

    @pl.when((b == 0) & (i == 0))
    def _init_all():
        focal_s[0, 0] = 0.0
        acc_s[0, 0] = 0.0

    @pl.when(i == 0)
    def _init_b():
        cmat_s[...] = jnp.zeros_like(cmat_s)
        intra_s[0, 0] = 0.0

    emb = emb_ref[0].reshape(_C, _P)
    inst = inst_ref[0, 0].reshape(1, _P)
    ids = lax.broadcasted_iota(jnp.int32, (_NI, 1), 0)
    maskf = (inst == ids).astype(jnp.float32)

    kf = kf_ref[0]
    kfsq = jnp.sum(kf * kf, axis=1, keepdims=True)
    kfa = jnp.concatenate([kf, kfsq], axis=1)
    gat = lax.dot_general(
        kfa, maskf, (((0,), (0,)), ((), ())),
        preferred_element_type=jnp.float32)
    kf_pix = gat[0:_C]
    na2 = gat[_C:_C + 1]
    dotp = jnp.sum(kf_pix * emb, axis=0, keepdims=True)
    nb2 = jnp.sum(emb * emb, axis=0, keepdims=True)
    na = jnp.sqrt(na2)
    nb = jnp.sqrt(nb2)
    cos = dotp / (jnp.maximum(na, 1e-8) * jnp.maximum(nb, 1e-8))
    intra_s[0, 0] += jnp.sum(jnp.exp(1.0 - cos) - 1.0)

    rect = rect_ref[0]
    x1 = rect[:, 0:1]
    x2 = rect[:, 1:2]
    y1 = rect[:, 2:3]
    y2 = rect[:, 3:4]
    xi = lax.broadcasted_iota(jnp.int32, (_NI, _W), 1).astype(jnp.float32)
    yi = (lax.broadcasted_iota(jnp.int32, (_NI, _HB), 1)
          + i * _HB).astype(jnp.float32)
    winx = ((xi >= x1) & (xi < x2)).astype(jnp.float32)
    winy = ((yi >= y1) & (yi < y2)).astype(jnp.float32)
    win = (winy[:, :, None] * winx[:, None, :]).reshape(_NI, _P)
    cmat_s[...] += lax.dot_general(
        win, maskf, (((1,), (1,)), ((), ())),
        preferred_element_type=jnp.float32)

    s0 = seed_ref[0, 0]
    s1 = seed_ref[0, 1]
    lbl = lbl_ref[0]
    m = jnp.maximum(s0, s1)
    lse = m + jnp.log(jnp.exp(s0 - m) + jnp.exp(s1 - m))
    lpt = jnp.where(lbl == 0, s0, s1) - lse
    pt = jnp.exp(lpt)
    focal_s[0, 0] += jnp.sum((1.0 - pt) * (1.0 - pt) * (-lpt))

    @pl.when(i == _NB - 1)
    def _fin():
        kfv = kf_ref[0]
        norms = jnp.maximum(
            jnp.sqrt(jnp.sum(kfv * kfv, axis=1, keepdims=True)), 1e-8)
        gram = lax.dot_general(
            kfv, kfv, (((1,), (1,)), ((), ())),
            preferred_element_type=jnp.float32)
        outer = lax.dot_general(
            norms, norms, (((1,), (1,)), ((), ())),
            preferred_element_type=jnp.float32)
        s_abs = jnp.abs(gram / outer)

        present = (cmat_s[...] > 0.0).astype(jnp.float32)
        ki = lax.broadcasted_iota(jnp.int32, (_NI, _NI), 0)
        ji = lax.broadcasted_iota(jnp.int32, (_NI, _NI), 1)
        upper = (ki > ji).astype(jnp.float32)
        suf = lax.dot_general(
            present, upper, (((1,), (0,)), ((), ())),
            preferred_element_type=jnp.float32)
        rowm = jnp.where((ji >= 1) & (suf <= 9.0), present, 0.0)
        nm = jnp.where((ki == 0) | (ji == 0), 0.5, rowm)
        nm = jnp.where((ki == 0) & (ji == 0), 0.0, nm)
        inter = jnp.sum((jnp.exp(s_abs) - 1.0) * nm) / jnp.sum(nm)

        acc_s[0, 0] += inter * _W_INST + \
            (intra_s[0, 0] / float(_H * _W)) * _W_VAR
        out_ref[0, 0] = focal_s[0, 0] / float(_B * _H * _W) * _W_SEED + \
            acc_s[0, 0] / float(_B)


def kernel(seed_maps, emb_maps, labels, instances):
    kf, rect = pl.pallas_call(
        _stats_body,
        grid=(_B, _NB),
        in_specs=[
            pl.BlockSpec((1, _C, _HB, _W), lambda b, i: (b, 0, i, 0)),
            pl.BlockSpec((1, 1, _HB, _W), lambda b, i: (b, 0, i, 0)),
        ],
        out_specs=[
            pl.BlockSpec((1, _NI, _C), lambda b, i: (b, 0, 0)),
            pl.BlockSpec((1, _NI, 4), lambda b, i: (b, 0, 0)),
        ],
        out_shape=[
            jax.ShapeDtypeStruct((_B, _NI, _C), jnp.float32),
            jax.ShapeDtypeStruct((_B, _NI, 4), jnp.float32),
        ],
        scratch_shapes=[
            pltpu.VMEM((_NI, _C), jnp.float32),
            pltpu.VMEM((_NI, 1), jnp.float32),
            pltpu.VMEM((_NI, 4), jnp.float32),
        ],
    )(emb_maps, instances)

    out = pl.pallas_call(
        _loss_body,
        grid=(_B, _NB),
        in_specs=[
            pl.BlockSpec((1, _C, _HB, _W), lambda b, i: (b, 0, i, 0)),
            pl.BlockSpec((1, 1, _HB, _W), lambda b, i: (b, 0, i, 0)),
            pl.BlockSpec((1, _NC, _HB, _W), lambda b, i: (b, 0, i, 0)),
            pl.BlockSpec((1, _HB, _W), lambda b, i: (b, i, 0)),
            pl.BlockSpec((1, _NI, _C), lambda b, i: (b, 0, 0)),
            pl.BlockSpec((1, _NI, 4), lambda b, i: (b, 0, 0)),
        ],
        out_specs=pl.BlockSpec(memory_space=pltpu.SMEM),
        out_shape=jax.ShapeDtypeStruct((1, 1), jnp.float32),
        scratch_shapes=[
            pltpu.VMEM((_NI, _NI), jnp.float32),
            pltpu.SMEM((1, 1), jnp.float32),
            pltpu.SMEM((1, 1), jnp.float32),
            pltpu.SMEM((1, 1), jnp.float32),
        ],
    )(emb_maps, instances, seed_maps, labels, kf, rect)

    return out[0, 0]
